# baseline (speedup 1.0000x reference)
import functools
import math

import jax
import jax.numpy as jnp
from jax import lax
from jax.experimental import pallas as pl
from jax.experimental.pallas import tpu as pltpu
from jax.experimental.pallas import tpu_sc as plsc

N_NODES = 10000
N_EDGES = 320000
D = 128
DH = 64
KHOPS = 5
NTILES = 16
NP = 10240
RPT = NP // NTILES
CH = 128
GR = 8
EPT = N_EDGES // NTILES
NCH = 160
EPAD = NCH * CH
NG = NCH // GR
PAD_IDX = NP - 1
NQ = RPT // CH


def _matmul_body(x_ref, w_ref, b_ref, out_ref):
    h = lax.dot_general(x_ref[...], w_ref[...],
                        (((1,), (1,)), ((), ())),
                        preferred_element_type=jnp.float32)
    h = h + b_ref[...]
    out_ref[0] = h[:, :DH]
    out_ref[1] = h[:, DH:]


def _matmul(xp, W, b2):
    BM = 1024
    return pl.pallas_call(
        _matmul_body,
        grid=(NP // BM,),
        in_specs=[
            pl.BlockSpec((BM, D), lambda i: (i, 0)),
            pl.BlockSpec((D, D), lambda i: (0, 0)),
            pl.BlockSpec((1, D), lambda i: (0, 0)),
        ],
        out_specs=pl.BlockSpec((2, BM, DH), lambda i: (0, i, 0)),
        out_shape=jax.ShapeDtypeStruct((2, NP, DH), jnp.float32),
    )(xp, W, b2)


def _diffuse_body(h2, src3, dst3, coef2, out_hbm, yh,
                  acc, degt,
                  src_c, dst_c, dinv_v, rows_v, sta_v, sty_v, sto_v, coef_v,
                  w16_v, sem, dsem):
    c = lax.axis_index("c")
    s = lax.axis_index("s")
    row0 = s * RPT
    bufs = (rows_v, sta_v, sty_v, sto_v)
    yv = yh.at[c]

    pltpu.sync_copy(coef2, coef_v)

    zeros16 = jnp.zeros((16,), dtype=jnp.float32)
    ones16 = jnp.full((16,), 1.0, dtype=jnp.float32)

    def fill_ones(r, _):
        w16_v[r, :] = ones16
        return _
    lax.fori_loop(0, CH, fill_ones, None)

    for q in range(NQ):
        pltpu.sync_copy(w16_v, degt.at[pl.ds(row0 + q * CH, CH)])
    plsc.subcore_barrier()

    pltpu.sync_copy(dst3.at[s, pl.ds(0, GR)], dst_c.at[0])

    def deg_gg(gg, _):
        for half in range(2):
            g = 2 * gg + half
            p = half
            for b in range(GR):
                pltpu.async_copy(w16_v, degt.at[dst_c.at[p, b]], dsem,
                                 add=True)

            @pl.when(g < NG - 1)
            def _stage(g=g, p=p):
                pltpu.sync_copy(dst3.at[s, pl.ds((g + 1) * GR, GR)],
                                dst_c.at[1 - p])
            for b in range(GR):
                pltpu.make_async_copy(w16_v, degt.at[dst_c.at[p, b]],
                                      dsem).wait()
        return _
    lax.fori_loop(0, NG // 2, deg_gg, None)
    plsc.subcore_barrier()

    for q in range(NQ):
        pltpu.sync_copy(degt.at[pl.ds(row0 + q * CH, CH)], w16_v)

        def rsq_row(r, _):
            d = w16_v[r, :]
            xf = ones16
            for _i in range(10):
                xf = jnp.where(xf * xf * d > 1.0, 0.5 * xf, xf)
            for _i in range(6):
                xf = xf * (1.5 - 0.5 * d * xf * xf)
            dinv_v[q * CH + r, :] = xf
            return _
        lax.fori_loop(0, CH, rsq_row, None)

    c0 = coef_v[0, :]
    for q in range(NQ):
        base = row0 + q * CH
        pltpu.sync_copy(h2.at[c, pl.ds(base, CH)], sty_v)

        def init_row(r, _):
            dv = dinv_v[q * CH + r, :]
            for j in range(DH // 16):
                sl = pl.ds(j * 16, 16)
                hv = sty_v[r, sl]
                sto_v[r, sl] = c0 * hv
                sty_v[r, sl] = dv * hv
                sta_v[r, sl] = zeros16
            return _
        lax.fori_loop(0, CH, init_row, None)
        pltpu.sync_copy(sty_v, yv.at[pl.ds(base, CH)])
        pltpu.sync_copy(sta_v, acc.at[pl.ds(base, CH)])
        pltpu.sync_copy(sto_v, out_hbm.at[c, pl.ds(base, CH)])
    plsc.subcore_barrier()

    for k in range(1, KHOPS + 1):
        pltpu.sync_copy(src3.at[s, pl.ds(0, GR)], src_c.at[0])
        pltpu.sync_copy(dst3.at[s, pl.ds(0, GR)], dst_c.at[0])

        def edge_gg(gg, _):
            for half in range(2):
                g = 2 * gg + half
                p = half
                j0 = g * GR
                for b in range(GR):
                    j = j0 + b
                    bf = b % 4

                    @pl.when(j >= 4)
                    def _wfree(p=p, b=b, bf=bf):
                        pltpu.make_async_copy(bufs[bf],
                                              acc.at[dst_c.at[p, b]],
                                              sem.at[bf]).wait()
                    pltpu.async_copy(yv.at[src_c.at[p, b]], bufs[bf],
                                     sem.at[bf])
                    if b >= 2:
                        pjj, bj = p, b - 2
                    else:
                        pjj, bj = 1 - p, b - 2 + GR
                    bf2 = (b - 2) % 4

                    @pl.when(j >= 2)
                    def _scat(pjj=pjj, bj=bj, bf2=bf2):
                        pltpu.make_async_copy(yv.at[src_c.at[pjj, bj]],
                                              bufs[bf2], sem.at[bf2]).wait()
                        pltpu.async_copy(bufs[bf2], acc.at[dst_c.at[pjj, bj]],
                                         sem.at[bf2], add=True)

                @pl.when(g < NG - 1)
                def _stage(g=g, p=p):
                    pltpu.sync_copy(src3.at[s, pl.ds((g + 1) * GR, GR)],
                                    src_c.at[1 - p])
                    pltpu.sync_copy(dst3.at[s, pl.ds((g + 1) * GR, GR)],
                                    dst_c.at[1 - p])
            return _
        lax.fori_loop(0, NG // 2, edge_gg, None)

        pl_last = (NG - 1) % 2
        for j in (NCH - 2, NCH - 1):
            bf = j % 4
            b = j % GR
            pltpu.make_async_copy(yv.at[src_c.at[pl_last, b]], bufs[bf],
                                  sem.at[bf]).wait()
            pltpu.async_copy(bufs[bf], acc.at[dst_c.at[pl_last, b]],
                             sem.at[bf], add=True)
        for j in range(NCH - 4, NCH):
            bf = j % 4
            pltpu.make_async_copy(bufs[bf], acc.at[dst_c.at[pl_last, j % GR]],
                                  sem.at[bf]).wait()
        plsc.subcore_barrier()

        ck = coef_v[k, :]
        last = (k == KHOPS)
        for q in range(NQ):
            base = row0 + q * CH
            pltpu.sync_copy(acc.at[pl.ds(base, CH)], sta_v)
            pltpu.sync_copy(yv.at[pl.ds(base, CH)], sty_v)
            pltpu.sync_copy(out_hbm.at[c, pl.ds(base, CH)], sto_v)

            def node_row(r, _):
                dv = dinv_v[q * CH + r, :]
                for j in range(DH // 16):
                    sl = pl.ds(j * 16, 16)
                    t1 = dv * (sta_v[r, sl] + sty_v[r, sl])
                    sto_v[r, sl] = sto_v[r, sl] + ck * t1
                    if not last:
                        sty_v[r, sl] = dv * t1
                        sta_v[r, sl] = zeros16
                return _
            lax.fori_loop(0, CH, node_row, None)
            pltpu.sync_copy(sto_v, out_hbm.at[c, pl.ds(base, CH)])
            if not last:
                pltpu.sync_copy(sty_v, yv.at[pl.ds(base, CH)])
                pltpu.sync_copy(sta_v, acc.at[pl.ds(base, CH)])
        if not last:
            plsc.subcore_barrier()


_diffuse = functools.partial(
    pl.kernel,
    out_type=[jax.ShapeDtypeStruct((2, NP, DH), jnp.float32),
              jax.ShapeDtypeStruct((2, NP, DH), jnp.float32)],
    mesh=plsc.VectorSubcoreMesh(core_axis_name="c", subcore_axis_name="s"),
    compiler_params=pltpu.CompilerParams(use_tc_tiling_on_sc=False),
    scratch_types=[
        pltpu.VMEM_SHARED((NP, DH), jnp.float32),
        pltpu.VMEM_SHARED((NP, 16), jnp.float32),
        pltpu.VMEM((2, GR, CH), jnp.int32),
        pltpu.VMEM((2, GR, CH), jnp.int32),
        pltpu.VMEM((RPT, 16), jnp.float32),
        pltpu.VMEM((CH, DH), jnp.float32),
        pltpu.VMEM((CH, DH), jnp.float32),
        pltpu.VMEM((CH, DH), jnp.float32),
        pltpu.VMEM((CH, DH), jnp.float32),
        pltpu.VMEM((8, 16), jnp.float32),
        pltpu.VMEM((CH, 16), jnp.float32),
        pltpu.SemaphoreType.DMA((4,)),
        pltpu.SemaphoreType.DMA,
    ],
)(_diffuse_body)


def kernel(x, edge_index, W, b, t):
    xp = jnp.pad(x, ((0, NP - N_NODES), (0, 0)))
    b2 = b.reshape(1, D)
    h2 = _matmul(xp, W, b2)

    ei = edge_index.astype(jnp.int32)
    src3 = jnp.pad(ei[0].reshape(NTILES, EPT), ((0, 0), (0, EPAD - EPT)),
                   constant_values=PAD_IDX).reshape(NTILES, NCH, CH)
    dst3 = jnp.pad(ei[1].reshape(NTILES, EPT), ((0, 0), (0, EPAD - EPT)),
                   constant_values=PAD_IDX).reshape(NTILES, NCH, CH)

    tf = t.astype(jnp.float32)
    coefs = jnp.stack([tf ** k / math.factorial(k) for k in range(KHOPS + 1)]
                      + [jnp.zeros_like(tf)] * (8 - (KHOPS + 1)))
    coef2 = (jnp.exp(-tf) * coefs)[:, None] * jnp.ones((1, 16), jnp.float32)

    out2, _ = _diffuse(h2, src3, dst3, coef2)
    return jnp.concatenate([out2[0], out2[1]], axis=1)[:N_NODES]


# speedup vs baseline: 1.2864x; 1.2864x over previous
import functools
import math

import jax
import jax.numpy as jnp
from jax import lax
from jax.experimental import pallas as pl
from jax.experimental.pallas import tpu as pltpu
from jax.experimental.pallas import tpu_sc as plsc

N_NODES = 10000
N_EDGES = 320000
D = 128
DH = 64
KHOPS = 5
NTILES = 16
NP = 10240
RPT = NP // NTILES
CH = 128
GR = 8
EPT = N_EDGES // NTILES
NCH = 160
EPAD = NCH * CH
NG = NCH // GR
PAD_IDX = NP - 1
NQ = RPT // CH


def _matmul_body(x_ref, w_ref, b_ref, out_ref):
    h = lax.dot_general(x_ref[...], w_ref[...],
                        (((1,), (1,)), ((), ())),
                        preferred_element_type=jnp.float32)
    h = h + b_ref[...]
    out_ref[0] = h[:, :DH]
    out_ref[1] = h[:, DH:]


def _matmul(xp, W, b2):
    BM = 1024
    return pl.pallas_call(
        _matmul_body,
        grid=(NP // BM,),
        in_specs=[
            pl.BlockSpec((BM, D), lambda i: (i, 0)),
            pl.BlockSpec((D, D), lambda i: (0, 0)),
            pl.BlockSpec((1, D), lambda i: (0, 0)),
        ],
        out_specs=pl.BlockSpec((2, BM, DH), lambda i: (0, i, 0)),
        out_shape=jax.ShapeDtypeStruct((2, NP, DH), jnp.float32),
    )(xp, W, b2)


def _diffuse_body(h2, src3, dst3, coef2, out_hbm,
                  ytab, acc,
                  src_c, dst_c, dinv_v, rows_v, sta_v, sty_v, sto_v, coef_v,
                  sem, dsem, sem2):
    c = lax.axis_index("c")
    s = lax.axis_index("s")
    row0 = s * RPT
    bufs = (rows_v, sta_v, sty_v, sto_v)

    pltpu.sync_copy(coef2, coef_v)

    zeros16 = jnp.zeros((16,), dtype=jnp.float32)
    ones16 = jnp.full((16,), 1.0, dtype=jnp.float32)

    def fill_ones(r, _):
        for j in range(DH // 16):
            rows_v[r, pl.ds(j * 16, 16)] = ones16
        return _
    lax.fori_loop(0, CH, fill_ones, None)

    for q in range(NQ):
        pltpu.sync_copy(rows_v, acc.at[pl.ds(row0 + q * CH, CH)])
    plsc.subcore_barrier()

    pltpu.async_copy(dst3.at[s, pl.ds(0, GR)], dst_c.at[0], sem2)

    def deg_gg(gg, _):
        for half in range(2):
            g = 2 * gg + half
            p = half
            pltpu.make_async_copy(dst3.at[s, pl.ds(0, GR)], dst_c.at[p],
                                  sem2).wait()

            @pl.when(g < NG - 1)
            def _stage(g=g, p=p):
                pltpu.async_copy(dst3.at[s, pl.ds((g + 1) * GR, GR)],
                                 dst_c.at[1 - p], sem2)
            for b in range(GR):
                pltpu.async_copy(rows_v, acc.at[dst_c.at[p, b]], dsem, add=True)
            for b in range(GR):
                pltpu.make_async_copy(rows_v, acc.at[dst_c.at[p, b]],
                                      dsem).wait()
        return _
    lax.fori_loop(0, NG // 2, deg_gg, None)
    plsc.subcore_barrier()

    for q in range(NQ):
        pltpu.sync_copy(acc.at[pl.ds(row0 + q * CH, CH)], sta_v)

        def rsq_row(r, _):
            d = sta_v[r, pl.ds(0, 16)]
            xf = ones16
            for _i in range(10):
                xf = jnp.where(xf * xf * d > 1.0, 0.5 * xf, xf)
            for _i in range(6):
                xf = xf * (1.5 - 0.5 * d * xf * xf)
            dinv_v[q * CH + r, :] = xf
            return _
        lax.fori_loop(0, CH, rsq_row, None)

    c0 = coef_v[0, :]
    for q in range(NQ):
        base = row0 + q * CH
        pltpu.sync_copy(h2.at[c, pl.ds(base, CH)], sty_v)

        def init_row(r, _):
            dv = dinv_v[q * CH + r, :]
            for j in range(DH // 16):
                sl = pl.ds(j * 16, 16)
                hv = sty_v[r, sl]
                sto_v[r, sl] = c0 * hv
                sty_v[r, sl] = dv * hv
                sta_v[r, sl] = zeros16
            return _
        lax.fori_loop(0, CH, init_row, None)
        pltpu.sync_copy(sty_v, ytab.at[pl.ds(base, CH)])
        pltpu.sync_copy(sta_v, acc.at[pl.ds(base, CH)])
        pltpu.sync_copy(sto_v, out_hbm.at[c, pl.ds(base, CH)])
    pltpu.async_copy(src3.at[s, pl.ds(0, GR)], src_c.at[0], sem2)
    pltpu.async_copy(dst3.at[s, pl.ds(0, GR)], dst_c.at[0], sem2)
    plsc.subcore_barrier()

    for k in range(1, KHOPS + 1):
        def edge_gg(gg, _):
            for half in range(2):
                g = 2 * gg + half
                p = half
                j0 = g * GR
                pltpu.make_async_copy(src3.at[s, pl.ds(0, GR)], src_c.at[p],
                                      sem2).wait()
                pltpu.make_async_copy(dst3.at[s, pl.ds(0, GR)], dst_c.at[p],
                                      sem2).wait()
                for b in range(GR):
                    j = j0 + b
                    bf = b % 4

                    @pl.when(j >= 4)
                    def _wfree(p=p, b=b, bf=bf):
                        pltpu.make_async_copy(bufs[bf],
                                              acc.at[dst_c.at[p, b]],
                                              sem.at[bf]).wait()
                    pltpu.async_copy(ytab.at[src_c.at[p, b]], bufs[bf],
                                     sem.at[bf])
                    if b >= 2:
                        pjj, bj = p, b - 2
                    else:
                        pjj, bj = 1 - p, b - 2 + GR
                    bf2 = (b - 2) % 4

                    @pl.when(j >= 2)
                    def _scat(pjj=pjj, bj=bj, bf2=bf2):
                        pltpu.make_async_copy(ytab.at[src_c.at[pjj, bj]],
                                              bufs[bf2], sem.at[bf2]).wait()
                        pltpu.async_copy(bufs[bf2], acc.at[dst_c.at[pjj, bj]],
                                         sem.at[bf2], add=True)
                    if b == 3:
                        @pl.when(g < NG - 1)
                        def _stage(g=g, p=p):
                            pltpu.async_copy(src3.at[s, pl.ds((g + 1) * GR, GR)],
                                             src_c.at[1 - p], sem2)
                            pltpu.async_copy(dst3.at[s, pl.ds((g + 1) * GR, GR)],
                                             dst_c.at[1 - p], sem2)
            return _
        lax.fori_loop(0, NG // 2, edge_gg, None)

        pl_last = (NG - 1) % 2
        for j in (NCH - 2, NCH - 1):
            bf = j % 4
            b = j % GR
            pltpu.make_async_copy(ytab.at[src_c.at[pl_last, b]], bufs[bf],
                                  sem.at[bf]).wait()
            pltpu.async_copy(bufs[bf], acc.at[dst_c.at[pl_last, b]],
                             sem.at[bf], add=True)
        for j in range(NCH - 4, NCH):
            bf = j % 4
            pltpu.make_async_copy(bufs[bf], acc.at[dst_c.at[pl_last, j % GR]],
                                  sem.at[bf]).wait()
        if k < KHOPS:
            pltpu.async_copy(src3.at[s, pl.ds(0, GR)], src_c.at[0], sem2)
            pltpu.async_copy(dst3.at[s, pl.ds(0, GR)], dst_c.at[0], sem2)
        plsc.subcore_barrier()

        ck = coef_v[k, :]
        last = (k == KHOPS)
        for q in range(NQ):
            base = row0 + q * CH
            pltpu.sync_copy(acc.at[pl.ds(base, CH)], sta_v)
            pltpu.sync_copy(ytab.at[pl.ds(base, CH)], sty_v)
            pltpu.sync_copy(out_hbm.at[c, pl.ds(base, CH)], sto_v)

            def node_row(r, _):
                dv = dinv_v[q * CH + r, :]
                for j in range(DH // 16):
                    sl = pl.ds(j * 16, 16)
                    t1 = dv * (sta_v[r, sl] + sty_v[r, sl])
                    sto_v[r, sl] = sto_v[r, sl] + ck * t1
                    if not last:
                        sty_v[r, sl] = dv * t1
                        sta_v[r, sl] = zeros16
                return _
            lax.fori_loop(0, CH, node_row, None)
            pltpu.sync_copy(sto_v, out_hbm.at[c, pl.ds(base, CH)])
            if not last:
                pltpu.sync_copy(sty_v, ytab.at[pl.ds(base, CH)])
                pltpu.sync_copy(sta_v, acc.at[pl.ds(base, CH)])
        if not last:
            plsc.subcore_barrier()


_diffuse = functools.partial(
    pl.kernel,
    out_type=jax.ShapeDtypeStruct((2, NP, DH), jnp.float32),
    mesh=plsc.VectorSubcoreMesh(core_axis_name="c", subcore_axis_name="s"),
    compiler_params=pltpu.CompilerParams(use_tc_tiling_on_sc=False),
    scratch_types=[
        pltpu.VMEM_SHARED((NP, DH), jnp.float32),
        pltpu.VMEM_SHARED((NP, DH), jnp.float32),
        pltpu.VMEM((2, GR, CH), jnp.int32),
        pltpu.VMEM((2, GR, CH), jnp.int32),
        pltpu.VMEM((RPT, 16), jnp.float32),
        pltpu.VMEM((CH, DH), jnp.float32),
        pltpu.VMEM((CH, DH), jnp.float32),
        pltpu.VMEM((CH, DH), jnp.float32),
        pltpu.VMEM((CH, DH), jnp.float32),
        pltpu.VMEM((8, 16), jnp.float32),
        pltpu.SemaphoreType.DMA((4,)),
        pltpu.SemaphoreType.DMA,
        pltpu.SemaphoreType.DMA,
    ],
)(_diffuse_body)


def kernel(x, edge_index, W, b, t):
    xp = jnp.pad(x, ((0, NP - N_NODES), (0, 0)))
    b2 = b.reshape(1, D)
    h2 = _matmul(xp, W, b2)

    ei = edge_index.astype(jnp.int32)
    src3 = jnp.pad(ei[0].reshape(NTILES, EPT), ((0, 0), (0, EPAD - EPT)),
                   constant_values=PAD_IDX).reshape(NTILES, NCH, CH)
    dst3 = jnp.pad(ei[1].reshape(NTILES, EPT), ((0, 0), (0, EPAD - EPT)),
                   constant_values=PAD_IDX).reshape(NTILES, NCH, CH)

    tf = t.astype(jnp.float32)
    coefs = jnp.stack([tf ** k / math.factorial(k) for k in range(KHOPS + 1)]
                      + [jnp.zeros_like(tf)] * (8 - (KHOPS + 1)))
    coef2 = (jnp.exp(-tf) * coefs)[:, None] * jnp.ones((1, 16), jnp.float32)

    out2 = _diffuse(h2, src3, dst3, coef2)
    return jnp.concatenate([out2[0], out2[1]], axis=1)[:N_NODES]


# speedup vs baseline: 1.7130x; 1.3316x over previous
import functools
import math

import jax
import jax.numpy as jnp
from jax import lax
from jax.experimental import pallas as pl
from jax.experimental.pallas import tpu as pltpu
from jax.experimental.pallas import tpu_sc as plsc

N_NODES = 10000
N_EDGES = 320000
D = 128
DH = 64
KHOPS = 5
NTILES = 16
NP = 10240
RPT = NP // NTILES
CH = 128
GR = 8
EPT = N_EDGES // NTILES
NCH = 160
EPAD = NCH * CH
NG = NCH // GR
PAD_IDX = NP - 1
NQ = RPT // CH


def _matmul_body(x_ref, w_ref, b_ref, out_ref):
    h = lax.dot_general(x_ref[...], w_ref[...],
                        (((1,), (1,)), ((), ())),
                        preferred_element_type=jnp.float32)
    h = h + b_ref[...]
    out_ref[0] = h[:, :DH]
    out_ref[1] = h[:, DH:]


def _matmul(xp, W, b2):
    BM = 1024
    return pl.pallas_call(
        _matmul_body,
        grid=(NP // BM,),
        in_specs=[
            pl.BlockSpec((BM, D), lambda i: (i, 0)),
            pl.BlockSpec((D, D), lambda i: (0, 0)),
            pl.BlockSpec((1, D), lambda i: (0, 0)),
        ],
        out_specs=pl.BlockSpec((2, BM, DH), lambda i: (0, i, 0)),
        out_shape=jax.ShapeDtypeStruct((2, NP, DH), jnp.float32),
    )(xp, W, b2)


def _diffuse_body(h2, src3, dst3, coef2, out_hbm,
                  ytab, acc,
                  src_c, dst_c, dinv_v, rows_v, sta_v, sty_v, sto_v, coef_v,
                  sem, dsem):
    c = lax.axis_index("c")
    s = lax.axis_index("s")
    row0 = s * RPT
    bufs = (rows_v, sta_v, sty_v, sto_v)

    pltpu.sync_copy(coef2, coef_v)

    zeros16 = jnp.zeros((16,), dtype=jnp.float32)
    ones16 = jnp.full((16,), 1.0, dtype=jnp.float32)

    def fill_ones(r, _):
        for j in range(DH // 16):
            rows_v[r, pl.ds(j * 16, 16)] = ones16
        return _
    lax.fori_loop(0, CH, fill_ones, None)

    for q in range(NQ):
        pltpu.sync_copy(rows_v, acc.at[pl.ds(row0 + q * CH, CH)])
    plsc.subcore_barrier()

    pltpu.sync_copy(dst3.at[s, pl.ds(0, GR)], dst_c.at[0])

    def deg_gg(gg, _):
        for half in range(2):
            g = 2 * gg + half
            p = half
            for b in range(GR):
                pltpu.async_copy(rows_v, acc.at[dst_c.at[p, b]], dsem, add=True)

            @pl.when(g < NG - 1)
            def _stage(g=g, p=p):
                pltpu.sync_copy(dst3.at[s, pl.ds((g + 1) * GR, GR)],
                                dst_c.at[1 - p])
            for b in range(GR):
                pltpu.make_async_copy(rows_v, acc.at[dst_c.at[p, b]],
                                      dsem).wait()
        return _
    lax.fori_loop(0, NG // 2, deg_gg, None)
    plsc.subcore_barrier()

    for q in range(NQ):
        pltpu.sync_copy(acc.at[pl.ds(row0 + q * CH, CH)], sta_v)

        def rsq_row(r, _):
            d = sta_v[r, pl.ds(0, 16)]
            xf = ones16
            for _i in range(10):
                xf = jnp.where(xf * xf * d > 1.0, 0.5 * xf, xf)
            for _i in range(6):
                xf = xf * (1.5 - 0.5 * d * xf * xf)
            dinv_v[q * CH + r, :] = xf
            return _
        lax.fori_loop(0, CH, rsq_row, None)

    c0 = coef_v[0, :]
    for q in range(NQ):
        base = row0 + q * CH
        pltpu.sync_copy(h2.at[c, pl.ds(base, CH)], sty_v)

        def init_row(r, _):
            dv = dinv_v[q * CH + r, :]
            for j in range(DH // 16):
                sl = pl.ds(j * 16, 16)
                hv = sty_v[r, sl]
                sto_v[r, sl] = c0 * hv
                sty_v[r, sl] = dv * hv
                sta_v[r, sl] = zeros16
            return _
        lax.fori_loop(0, CH, init_row, None)
        pltpu.sync_copy(sty_v, ytab.at[pl.ds(base, CH)])
        pltpu.sync_copy(sta_v, acc.at[pl.ds(base, CH)])
        pltpu.sync_copy(sto_v, out_hbm.at[c, pl.ds(base, CH)])
    plsc.subcore_barrier()

    for k in range(1, KHOPS + 1):
        pltpu.sync_copy(src3.at[s, pl.ds(0, GR)], src_c.at[0])
        pltpu.sync_copy(dst3.at[s, pl.ds(0, GR)], dst_c.at[0])

        def edge_gg(gg, _):
            for half in range(2):
                g = 2 * gg + half
                p = half
                j0 = g * GR
                for b in range(GR):
                    j = j0 + b
                    bf = b % 4

                    @pl.when(j >= 4)
                    def _wfree(p=p, b=b, bf=bf):
                        pltpu.make_async_copy(bufs[bf],
                                              acc.at[dst_c.at[p, b]],
                                              sem.at[bf]).wait()
                    pltpu.async_copy(ytab.at[src_c.at[p, b]], bufs[bf],
                                     sem.at[bf])
                    if b >= 2:
                        pjj, bj = p, b - 2
                    else:
                        pjj, bj = 1 - p, b - 2 + GR
                    bf2 = (b - 2) % 4

                    @pl.when(j >= 2)
                    def _scat(pjj=pjj, bj=bj, bf2=bf2):
                        pltpu.make_async_copy(ytab.at[src_c.at[pjj, bj]],
                                              bufs[bf2], sem.at[bf2]).wait()
                        pltpu.async_copy(bufs[bf2], acc.at[dst_c.at[pjj, bj]],
                                         sem.at[bf2], add=True)

                @pl.when(g < NG - 1)
                def _stage(g=g, p=p):
                    pltpu.sync_copy(src3.at[s, pl.ds((g + 1) * GR, GR)],
                                    src_c.at[1 - p])
                    pltpu.sync_copy(dst3.at[s, pl.ds((g + 1) * GR, GR)],
                                    dst_c.at[1 - p])
            return _
        lax.fori_loop(0, NG // 2, edge_gg, None)

        pl_last = (NG - 1) % 2
        for j in (NCH - 2, NCH - 1):
            bf = j % 4
            b = j % GR
            pltpu.make_async_copy(ytab.at[src_c.at[pl_last, b]], bufs[bf],
                                  sem.at[bf]).wait()
            pltpu.async_copy(bufs[bf], acc.at[dst_c.at[pl_last, b]],
                             sem.at[bf], add=True)
        for j in range(NCH - 4, NCH):
            bf = j % 4
            pltpu.make_async_copy(bufs[bf], acc.at[dst_c.at[pl_last, j % GR]],
                                  sem.at[bf]).wait()
        plsc.subcore_barrier()

        ck = coef_v[k, :]
        last = (k == KHOPS)
        for q in range(NQ):
            base = row0 + q * CH
            pltpu.sync_copy(acc.at[pl.ds(base, CH)], sta_v)
            pltpu.sync_copy(ytab.at[pl.ds(base, CH)], sty_v)
            pltpu.sync_copy(out_hbm.at[c, pl.ds(base, CH)], sto_v)

            def node_row(r, _):
                dv = dinv_v[q * CH + r, :]
                for j in range(DH // 16):
                    sl = pl.ds(j * 16, 16)
                    t1 = dv * (sta_v[r, sl] + sty_v[r, sl])
                    sto_v[r, sl] = sto_v[r, sl] + ck * t1
                    if not last:
                        sty_v[r, sl] = dv * t1
                        sta_v[r, sl] = zeros16
                return _
            lax.fori_loop(0, CH, node_row, None)
            pltpu.sync_copy(sto_v, out_hbm.at[c, pl.ds(base, CH)])
            if not last:
                pltpu.sync_copy(sty_v, ytab.at[pl.ds(base, CH)])
                pltpu.sync_copy(sta_v, acc.at[pl.ds(base, CH)])
        if not last:
            plsc.subcore_barrier()


_diffuse = functools.partial(
    pl.kernel,
    out_type=jax.ShapeDtypeStruct((2, NP, DH), jnp.float32),
    mesh=plsc.VectorSubcoreMesh(core_axis_name="c", subcore_axis_name="s"),
    compiler_params=pltpu.CompilerParams(use_tc_tiling_on_sc=False),
    scratch_types=[
        pltpu.VMEM_SHARED((NP, DH), jnp.float32),
        pltpu.VMEM_SHARED((NP, DH), jnp.float32),
        pltpu.VMEM((2, GR, CH), jnp.int32),
        pltpu.VMEM((2, GR, CH), jnp.int32),
        pltpu.VMEM((RPT, 16), jnp.float32),
        pltpu.VMEM((CH, DH), jnp.float32),
        pltpu.VMEM((CH, DH), jnp.float32),
        pltpu.VMEM((CH, DH), jnp.float32),
        pltpu.VMEM((CH, DH), jnp.float32),
        pltpu.VMEM((8, 16), jnp.float32),
        pltpu.SemaphoreType.DMA((4,)),
        pltpu.SemaphoreType.DMA,
    ],
)(_diffuse_body)


def kernel(x, edge_index, W, b, t):
    xp = jnp.pad(x, ((0, NP - N_NODES), (0, 0)))
    b2 = b.reshape(1, D)
    h2 = _matmul(xp, W, b2)

    ei = edge_index.astype(jnp.int32)
    src3 = jnp.pad(ei[0].reshape(NTILES, EPT), ((0, 0), (0, EPAD - EPT)),
                   constant_values=PAD_IDX).reshape(NTILES, NCH, CH)
    dst3 = jnp.pad(ei[1].reshape(NTILES, EPT), ((0, 0), (0, EPAD - EPT)),
                   constant_values=PAD_IDX).reshape(NTILES, NCH, CH)

    tf = t.astype(jnp.float32)
    coefs = jnp.stack([tf ** k / math.factorial(k) for k in range(KHOPS + 1)]
                      + [jnp.zeros_like(tf)] * (8 - (KHOPS + 1)))
    coef2 = (jnp.exp(-tf) * coefs)[:, None] * jnp.ones((1, 16), jnp.float32)

    out2 = _diffuse(h2, src3, dst3, coef2)
    return jnp.concatenate([out2[0], out2[1]], axis=1)[:N_NODES]
